# Initial kernel scaffold; baseline (speedup 1.0000x reference)
import jax
import jax.numpy as jnp
from jax.experimental import pallas as pl


def kernel(x, edge_index, batch, W0, b0, gnw0, gnb0, gnm0, W1, b1, gnw1, gnb1, gnm1, W2, b2, gnw2, gnb2, gnm2, Wd1, bd1, Wd2, bd2):
    raise NotImplementedError("write your pallas kernel here")


# speedup vs baseline: 14.7452x; 14.7452x over previous
import functools

import jax
import jax.numpy as jnp
from jax import lax
from jax.experimental import pallas as pl
from jax.experimental.pallas import tpu as pltpu
from jax.experimental.pallas import tpu_sc as plsc

_N = 10000
_E = 320000
_D = 128
_G = 64
_C = 32

_NP = 10240
_K = 80
_NW = 32
_EPW = _E // _NW
_CH = _EPW // _K
_RPS = _NP // 16

_R = 2000
_GRID = _N // _R

_HIGH = lax.Precision.HIGHEST


def _dot(a, b):
    return lax.dot_general(a, b, (((1,), (0,)), ((), ())), precision=_HIGH)


def _dot_t(a, b):
    return lax.dot_general(a, b, (((0,), (0,)), ((), ())), precision=_HIGH)


def _sc_mesh():
    return plsc.VectorSubcoreMesh(core_axis_name="c", subcore_axis_name="s")


def _deg_body(dst_hbm, zeros_hbm, ones_hbm, out_hbm, idx_d, ones_v, acc):
    c = lax.axis_index("c")
    s = lax.axis_index("s")
    w = c * 16 + s
    r0 = s * _RPS
    pltpu.sync_copy(ones_hbm, ones_v)
    pltpu.sync_copy(zeros_hbm.at[pl.ds(r0, _RPS)], acc.at[pl.ds(r0, _RPS)])
    pltpu.sync_copy(dst_hbm.at[w], idx_d)
    plsc.subcore_barrier()

    def chunk(j, carry):
        pltpu.sync_copy(ones_v, acc.at[idx_d.at[j]], add=True)
        return carry

    lax.fori_loop(0, _CH, chunk, 0)
    plsc.subcore_barrier()
    pltpu.sync_copy(acc.at[pl.ds(r0, _RPS)], out_hbm.at[c, pl.ds(r0, _RPS)])


def _deg_call(dst2d, zeros128, ones128):
    kern = pl.kernel(
        _deg_body,
        out_type=jax.ShapeDtypeStruct((2, _NP, _D), jnp.float32),
        mesh=_sc_mesh(),
        scratch_types=[
            pltpu.VMEM((_CH, _K), jnp.int32),
            pltpu.VMEM((_K, _D), jnp.float32),
            pltpu.VMEM_SHARED((_NP, _D), jnp.float32),
        ],
    )
    return kern(dst2d, zeros128, ones128)


def _scatter_body(h_hbm, src_hbm, dst_hbm, zeros_hbm, out_hbm,
                  idx_s, idx_d, rows, acc, sem):
    c = lax.axis_index("c")
    s = lax.axis_index("s")
    w = c * 16 + s
    r0 = s * _RPS
    pltpu.sync_copy(zeros_hbm.at[pl.ds(r0, _RPS)], acc.at[pl.ds(r0, _RPS)])
    pltpu.sync_copy(src_hbm.at[w], idx_s)
    pltpu.sync_copy(dst_hbm.at[w], idx_d)
    plsc.subcore_barrier()

    def chunk(j, carry):
        pltpu.async_copy(h_hbm.at[idx_s.at[j]], rows, sem).wait()
        pltpu.sync_copy(rows, acc.at[idx_d.at[j]], add=True)
        return carry

    lax.fori_loop(0, _CH, chunk, 0)
    plsc.subcore_barrier()
    pltpu.sync_copy(acc.at[pl.ds(r0, _RPS)], out_hbm.at[c, pl.ds(r0, _RPS)])


def _scatter_call(h, src2d, dst2d, zeros128):
    kern = pl.kernel(
        _scatter_body,
        out_type=jax.ShapeDtypeStruct((2, _NP, _D), jnp.float32),
        mesh=_sc_mesh(),
        scratch_types=[
            pltpu.VMEM((_CH, _K), jnp.int32),
            pltpu.VMEM((_CH, _K), jnp.int32),
            pltpu.VMEM((_K, _D), jnp.float32),
            pltpu.VMEM_SHARED((_NP, _D), jnp.float32),
            pltpu.SemaphoreType.DMA,
        ],
    )
    return kern(h, src2d, dst2d, zeros128)


def _prep_kernel(degp_ref, x_ref, w_ref, dinv_ref, ht_ref):
    deg = degp_ref[0, :, 0:1] + degp_ref[1, :, 0:1] + 1.0
    dinv = lax.rsqrt(deg)
    dinv_ref[...] = dinv
    ht_ref[...] = _dot(x_ref[...], w_ref[...]) * dinv


def _prep_call(x, W0, degp):
    return pl.pallas_call(
        _prep_kernel,
        grid=(_GRID,),
        in_specs=[
            pl.BlockSpec((2, _R, _D), lambda i: (0, i, 0)),
            pl.BlockSpec((_R, _D), lambda i: (i, 0)),
            pl.BlockSpec((_D, _D), lambda i: (0, 0)),
        ],
        out_specs=[
            pl.BlockSpec((_R, 1), lambda i: (i, 0)),
            pl.BlockSpec((_R, _D), lambda i: (i, 0)),
        ],
        out_shape=[
            jax.ShapeDtypeStruct((_N, 1), jnp.float32),
            jax.ShapeDtypeStruct((_N, _D), jnp.float32),
        ],
    )(degp, x, W0)


def _stats_kernel(parts_ref, ht_ref, dinv_ref, batch_ref, b_ref,
                  t_ref, s_ref, s_acc):
    i = pl.program_id(0)
    t = dinv_ref[...] * (parts_ref[0] + parts_ref[1] + ht_ref[...]) + b_ref[...]
    t_ref[...] = t
    onehot = (batch_ref[...] ==
              lax.broadcasted_iota(jnp.int32, (_R, _G), 1)).astype(jnp.float32)
    s0 = _dot_t(onehot, jnp.ones_like(t))
    s1 = _dot_t(onehot, t)
    s2 = _dot_t(onehot, t * t)
    blk = jnp.stack([s0, s1, s2])

    @pl.when(i == 0)
    def _():
        s_acc[...] = blk

    @pl.when(i > 0)
    def _():
        s_acc[...] += blk

    @pl.when(i == _GRID - 1)
    def _():
        s_ref[...] = s_acc[...]


def _stats_call(parts, ht, dinv, batch2d, b):
    return pl.pallas_call(
        _stats_kernel,
        grid=(_GRID,),
        in_specs=[
            pl.BlockSpec((2, _R, _D), lambda i: (0, i, 0)),
            pl.BlockSpec((_R, _D), lambda i: (i, 0)),
            pl.BlockSpec((_R, 1), lambda i: (i, 0)),
            pl.BlockSpec((_R, 1), lambda i: (i, 0)),
            pl.BlockSpec((1, _D), lambda i: (0, 0)),
        ],
        out_specs=[
            pl.BlockSpec((_R, _D), lambda i: (i, 0)),
            pl.BlockSpec((3, _G, _D), lambda i: (0, 0, 0)),
        ],
        out_shape=[
            jax.ShapeDtypeStruct((_N, _D), jnp.float32),
            jax.ShapeDtypeStruct((3, _G, _D), jnp.float32),
        ],
        scratch_shapes=[pltpu.VMEM((3, _G, _D), jnp.float32)],
    )(parts, ht, dinv, batch2d, b.reshape(1, _D))


def _norm_kernel(t_ref, s_ref, batch_ref, dinv_ref, gw_ref, gb_ref, gm_ref,
                 wn_ref, emb_ref, htn_ref, pool_ref, pool_acc):
    i = pl.program_id(0)
    cnt = jnp.maximum(s_ref[0], 1.0)
    mean = s_ref[1] / cnt
    ms = gm_ref[...]
    var = s_ref[2] / cnt - (2.0 * ms - ms * ms) * mean * mean
    inv_std = lax.rsqrt(var + 1e-5)
    onehot = (batch_ref[...] ==
              lax.broadcasted_iota(jnp.int32, (_R, _G), 1)).astype(jnp.float32)
    mean_b = _dot(onehot, mean * ms)
    istd_b = _dot(onehot, inv_std)
    t = t_ref[...]
    h = jnp.maximum((t - mean_b) * istd_b * gw_ref[...] + gb_ref[...], 0.0)
    emb_ref[...] = h
    htn_ref[...] = _dot(h, wn_ref[...]) * dinv_ref[...]
    blk = _dot_t(onehot, h)

    @pl.when(i == 0)
    def _():
        pool_acc[...] = blk

    @pl.when(i > 0)
    def _():
        pool_acc[...] += blk

    @pl.when(i == _GRID - 1)
    def _():
        pool_ref[...] = pool_acc[...]


def _norm_call(t, S, batch2d, dinv, gw, gb, gm, Wn):
    return pl.pallas_call(
        _norm_kernel,
        grid=(_GRID,),
        in_specs=[
            pl.BlockSpec((_R, _D), lambda i: (i, 0)),
            pl.BlockSpec((3, _G, _D), lambda i: (0, 0, 0)),
            pl.BlockSpec((_R, 1), lambda i: (i, 0)),
            pl.BlockSpec((_R, 1), lambda i: (i, 0)),
            pl.BlockSpec((1, _D), lambda i: (0, 0)),
            pl.BlockSpec((1, _D), lambda i: (0, 0)),
            pl.BlockSpec((1, _D), lambda i: (0, 0)),
            pl.BlockSpec((_D, _D), lambda i: (0, 0)),
        ],
        out_specs=[
            pl.BlockSpec((_R, _D), lambda i: (i, 0)),
            pl.BlockSpec((_R, _D), lambda i: (i, 0)),
            pl.BlockSpec((_G, _D), lambda i: (0, 0)),
        ],
        out_shape=[
            jax.ShapeDtypeStruct((_N, _D), jnp.float32),
            jax.ShapeDtypeStruct((_N, _D), jnp.float32),
            jax.ShapeDtypeStruct((_G, _D), jnp.float32),
        ],
        scratch_shapes=[pltpu.VMEM((_G, _D), jnp.float32)],
    )(t, S, batch2d, dinv, gw.reshape(1, _D), gb.reshape(1, _D),
      gm.reshape(1, _D), Wn)


def _head_kernel(pools_ref, s0_ref, wd1_ref, bd1_ref, wd2_ref, bd2_ref, z_ref):
    cnt = jnp.maximum(s0_ref[:, 0:1], 1.0)
    pm = jnp.concatenate(
        [pools_ref[0] / cnt, pools_ref[1] / cnt, pools_ref[2] / cnt], axis=1)
    z1 = jnp.maximum(_dot(pm, wd1_ref[...]) + bd1_ref[...], 0.0)
    z2 = _dot(z1, wd2_ref[...]) + bd2_ref[...]
    m = jnp.max(z2, axis=1, keepdims=True)
    e = z2 - m
    z_ref[...] = e - jnp.log(jnp.sum(jnp.exp(e), axis=1, keepdims=True))


def _head_call(pools, S0, Wd1, bd1, Wd2, bd2):
    hd = _D * 3
    return pl.pallas_call(
        _head_kernel,
        out_shape=jax.ShapeDtypeStruct((_G, _C), jnp.float32),
    )(pools, S0, Wd1, bd1.reshape(1, hd), Wd2, bd2.reshape(1, _C))


@jax.jit
def kernel(x, edge_index, batch, W0, b0, gnw0, gnb0, gnm0, W1, b1, gnw1, gnb1,
           gnm1, W2, b2, gnw2, gnb2, gnm2, Wd1, bd1, Wd2, bd2):
    src2d = edge_index[0].astype(jnp.int32).reshape(_NW, _CH, _K)
    dst2d = edge_index[1].astype(jnp.int32).reshape(_NW, _CH, _K)
    batch2d = batch.astype(jnp.int32).reshape(_N, 1)
    zeros128 = jnp.zeros((_NP, _D), jnp.float32)
    ones128 = jnp.ones((_K, _D), jnp.float32)

    degp = _deg_call(dst2d, zeros128, ones128)
    dinv, ht = _prep_call(x, W0, degp)

    layers = [(b0, gnw0, gnb0, gnm0, W1),
              (b1, gnw1, gnb1, gnm1, W2),
              (b2, gnw2, gnb2, gnm2, jnp.zeros((_D, _D), jnp.float32))]
    pools = []
    S0_saved = None
    emb = None
    for (b, gw, gb, gm, Wn) in layers:
        parts = _scatter_call(ht, src2d, dst2d, zeros128)
        t, S = _stats_call(parts, ht, dinv, batch2d, b)
        if S0_saved is None:
            S0_saved = S[0]
        emb, ht, pool = _norm_call(t, S, batch2d, dinv, gw, gb, gm, Wn)
        pools.append(pool)

    z = _head_call(jnp.stack(pools), S0_saved, Wd1, bd1, Wd2, bd2)
    return (emb, z)


# speedup vs baseline: 21.1823x; 1.4366x over previous
import functools

import jax
import jax.numpy as jnp
from jax import lax
from jax.experimental import pallas as pl
from jax.experimental.pallas import tpu as pltpu
from jax.experimental.pallas import tpu_sc as plsc

_N = 10000
_E = 320000
_D = 128
_G = 64
_C = 32

_NP = 10240
_K = 80
_NW = 32
_EPW = _E // _NW
_CH = _EPW // _K
_RPS = _NP // 16

_R = 2000
_GRID = _N // _R

_HIGH = lax.Precision.HIGHEST


def _dot(a, b):
    return lax.dot_general(a, b, (((1,), (0,)), ((), ())), precision=_HIGH)


def _dot_t(a, b):
    return lax.dot_general(a, b, (((0,), (0,)), ((), ())), precision=_HIGH)


def _sc_mesh():
    return plsc.VectorSubcoreMesh(core_axis_name="c", subcore_axis_name="s")


def _deg_body(dst_hbm, zeros_hbm, ones_hbm, out_hbm, idx_d, ones_v, acc):
    c = lax.axis_index("c")
    s = lax.axis_index("s")
    w = c * 16 + s
    r0 = s * _RPS
    pltpu.sync_copy(ones_hbm, ones_v)
    pltpu.sync_copy(zeros_hbm.at[pl.ds(r0, _RPS)], acc.at[pl.ds(r0, _RPS)])
    pltpu.sync_copy(dst_hbm.at[w], idx_d)
    plsc.subcore_barrier()

    def chunk(j, carry):
        pltpu.sync_copy(ones_v, acc.at[idx_d.at[j]], add=True)
        return carry

    lax.fori_loop(0, _CH, chunk, 0)
    plsc.subcore_barrier()
    pltpu.sync_copy(acc.at[pl.ds(r0, _RPS)], out_hbm.at[c, pl.ds(r0, _RPS)])


def _deg_call(dst2d, zeros128, ones128):
    kern = pl.kernel(
        _deg_body,
        out_type=jax.ShapeDtypeStruct((2, _NP, _D), jnp.float32),
        mesh=_sc_mesh(),
        scratch_types=[
            pltpu.VMEM((_CH, _K), jnp.int32),
            pltpu.VMEM((_K, _D), jnp.float32),
            pltpu.VMEM_SHARED((_NP, _D), jnp.float32),
        ],
    )
    return kern(dst2d, zeros128, ones128)


def _scatter_body(h_hbm, src_hbm, dst_hbm, zeros_hbm, out_hbm,
                  idx_s, idx_d, rows0, rows1, acc, sem0, sem1):
    c = lax.axis_index("c")
    s = lax.axis_index("s")
    w = c * 16 + s
    r0 = s * _RPS
    pltpu.sync_copy(zeros_hbm.at[pl.ds(r0, _RPS)], acc.at[pl.ds(r0, _RPS)])
    pltpu.sync_copy(src_hbm.at[pl.ds(w * _EPW, _EPW)], idx_s)
    pltpu.sync_copy(dst_hbm.at[w], idx_d)
    plsc.subcore_barrier()

    def gat(j, buf, sem):
        pltpu.async_copy(h_hbm.at[idx_s.at[pl.ds(j * _K, _K)]], buf, sem)

    def gat_wait(j, buf, sem):
        pltpu.make_async_copy(h_hbm.at[idx_s.at[pl.ds(j * _K, _K)]], buf,
                              sem).wait()

    gat(0, rows0, sem0)

    def body(jj, carry):
        j = jj * 2
        gat(j + 1, rows1, sem1)
        gat_wait(j, rows0, sem0)
        pltpu.sync_copy(rows0, acc.at[idx_d.at[j]], add=True)
        gat(j + 2, rows0, sem0)
        gat_wait(j + 1, rows1, sem1)
        pltpu.sync_copy(rows1, acc.at[idx_d.at[j + 1]], add=True)
        return carry

    lax.fori_loop(0, (_CH - 1) // 2, body, 0)
    gat_wait(_CH - 1, rows0, sem0)
    pltpu.sync_copy(rows0, acc.at[idx_d.at[_CH - 1]], add=True)
    plsc.subcore_barrier()
    pltpu.sync_copy(acc.at[pl.ds(r0, _RPS)], out_hbm.at[c, pl.ds(r0, _RPS)])


def _scatter_call(h, src2d, dst2d, zeros128):
    kern = pl.kernel(
        _scatter_body,
        out_type=jax.ShapeDtypeStruct((2, _NP, _D), jnp.float32),
        mesh=_sc_mesh(),
        scratch_types=[
            pltpu.VMEM((_EPW,), jnp.int32),
            pltpu.VMEM((_CH, _K), jnp.int32),
            pltpu.VMEM((_K, _D), jnp.float32),
            pltpu.VMEM((_K, _D), jnp.float32),
            pltpu.VMEM_SHARED((_NP, _D), jnp.float32),
            pltpu.SemaphoreType.DMA,
            pltpu.SemaphoreType.DMA,
        ],
    )
    return kern(h, src2d, dst2d, zeros128)


def _prep_kernel(degp_ref, x_ref, w_ref, dinv_ref, ht_ref):
    deg = degp_ref[0, :, 0:1] + degp_ref[1, :, 0:1] + 1.0
    dinv = lax.rsqrt(deg)
    dinv_ref[...] = dinv
    ht_ref[...] = _dot(x_ref[...], w_ref[...]) * dinv


def _prep_call(x, W0, degp):
    return pl.pallas_call(
        _prep_kernel,
        grid=(_GRID,),
        in_specs=[
            pl.BlockSpec((2, _R, _D), lambda i: (0, i, 0)),
            pl.BlockSpec((_R, _D), lambda i: (i, 0)),
            pl.BlockSpec((_D, _D), lambda i: (0, 0)),
        ],
        out_specs=[
            pl.BlockSpec((_R, 1), lambda i: (i, 0)),
            pl.BlockSpec((_R, _D), lambda i: (i, 0)),
        ],
        out_shape=[
            jax.ShapeDtypeStruct((_N, 1), jnp.float32),
            jax.ShapeDtypeStruct((_N, _D), jnp.float32),
        ],
    )(degp, x, W0)


def _stats_kernel(parts_ref, ht_ref, dinv_ref, batch_ref, b_ref,
                  t_ref, s_ref, s_acc):
    i = pl.program_id(0)
    t = dinv_ref[...] * (parts_ref[0] + parts_ref[1] + ht_ref[...]) + b_ref[...]
    t_ref[...] = t
    onehot = (batch_ref[...] ==
              lax.broadcasted_iota(jnp.int32, (_R, _G), 1)).astype(jnp.float32)
    s0 = _dot_t(onehot, jnp.ones_like(t))
    s1 = _dot_t(onehot, t)
    s2 = _dot_t(onehot, t * t)
    blk = jnp.stack([s0, s1, s2])

    @pl.when(i == 0)
    def _():
        s_acc[...] = blk

    @pl.when(i > 0)
    def _():
        s_acc[...] += blk

    @pl.when(i == _GRID - 1)
    def _():
        s_ref[...] = s_acc[...]


def _stats_call(parts, ht, dinv, batch2d, b):
    return pl.pallas_call(
        _stats_kernel,
        grid=(_GRID,),
        in_specs=[
            pl.BlockSpec((2, _R, _D), lambda i: (0, i, 0)),
            pl.BlockSpec((_R, _D), lambda i: (i, 0)),
            pl.BlockSpec((_R, 1), lambda i: (i, 0)),
            pl.BlockSpec((_R, 1), lambda i: (i, 0)),
            pl.BlockSpec((1, _D), lambda i: (0, 0)),
        ],
        out_specs=[
            pl.BlockSpec((_R, _D), lambda i: (i, 0)),
            pl.BlockSpec((3, _G, _D), lambda i: (0, 0, 0)),
        ],
        out_shape=[
            jax.ShapeDtypeStruct((_N, _D), jnp.float32),
            jax.ShapeDtypeStruct((3, _G, _D), jnp.float32),
        ],
        scratch_shapes=[pltpu.VMEM((3, _G, _D), jnp.float32)],
    )(parts, ht, dinv, batch2d, b.reshape(1, _D))


def _norm_kernel(t_ref, s_ref, batch_ref, dinv_ref, gw_ref, gb_ref, gm_ref,
                 wn_ref, emb_ref, htn_ref, pool_ref, pool_acc):
    i = pl.program_id(0)
    cnt = jnp.maximum(s_ref[0], 1.0)
    mean = s_ref[1] / cnt
    ms = gm_ref[...]
    var = s_ref[2] / cnt - (2.0 * ms - ms * ms) * mean * mean
    inv_std = lax.rsqrt(var + 1e-5)
    onehot = (batch_ref[...] ==
              lax.broadcasted_iota(jnp.int32, (_R, _G), 1)).astype(jnp.float32)
    mean_b = _dot(onehot, mean * ms)
    istd_b = _dot(onehot, inv_std)
    t = t_ref[...]
    h = jnp.maximum((t - mean_b) * istd_b * gw_ref[...] + gb_ref[...], 0.0)
    emb_ref[...] = h
    htn_ref[...] = _dot(h, wn_ref[...]) * dinv_ref[...]
    blk = _dot_t(onehot, h)

    @pl.when(i == 0)
    def _():
        pool_acc[...] = blk

    @pl.when(i > 0)
    def _():
        pool_acc[...] += blk

    @pl.when(i == _GRID - 1)
    def _():
        pool_ref[...] = pool_acc[...]


def _norm_call(t, S, batch2d, dinv, gw, gb, gm, Wn):
    return pl.pallas_call(
        _norm_kernel,
        grid=(_GRID,),
        in_specs=[
            pl.BlockSpec((_R, _D), lambda i: (i, 0)),
            pl.BlockSpec((3, _G, _D), lambda i: (0, 0, 0)),
            pl.BlockSpec((_R, 1), lambda i: (i, 0)),
            pl.BlockSpec((_R, 1), lambda i: (i, 0)),
            pl.BlockSpec((1, _D), lambda i: (0, 0)),
            pl.BlockSpec((1, _D), lambda i: (0, 0)),
            pl.BlockSpec((1, _D), lambda i: (0, 0)),
            pl.BlockSpec((_D, _D), lambda i: (0, 0)),
        ],
        out_specs=[
            pl.BlockSpec((_R, _D), lambda i: (i, 0)),
            pl.BlockSpec((_R, _D), lambda i: (i, 0)),
            pl.BlockSpec((_G, _D), lambda i: (0, 0)),
        ],
        out_shape=[
            jax.ShapeDtypeStruct((_N, _D), jnp.float32),
            jax.ShapeDtypeStruct((_N, _D), jnp.float32),
            jax.ShapeDtypeStruct((_G, _D), jnp.float32),
        ],
        scratch_shapes=[pltpu.VMEM((_G, _D), jnp.float32)],
    )(t, S, batch2d, dinv, gw.reshape(1, _D), gb.reshape(1, _D),
      gm.reshape(1, _D), Wn)


def _head_kernel(pools_ref, s0_ref, wd1_ref, bd1_ref, wd2_ref, bd2_ref, z_ref):
    cnt = jnp.maximum(s0_ref[:, 0:1], 1.0)
    pm = jnp.concatenate(
        [pools_ref[0] / cnt, pools_ref[1] / cnt, pools_ref[2] / cnt], axis=1)
    z1 = jnp.maximum(_dot(pm, wd1_ref[...]) + bd1_ref[...], 0.0)
    z2 = _dot(z1, wd2_ref[...]) + bd2_ref[...]
    m = jnp.max(z2, axis=1, keepdims=True)
    e = z2 - m
    z_ref[...] = e - jnp.log(jnp.sum(jnp.exp(e), axis=1, keepdims=True))


def _head_call(pools, S0, Wd1, bd1, Wd2, bd2):
    hd = _D * 3
    return pl.pallas_call(
        _head_kernel,
        out_shape=jax.ShapeDtypeStruct((_G, _C), jnp.float32),
    )(pools, S0, Wd1, bd1.reshape(1, hd), Wd2, bd2.reshape(1, _C))


@jax.jit
def kernel(x, edge_index, batch, W0, b0, gnw0, gnb0, gnm0, W1, b1, gnw1, gnb1,
           gnm1, W2, b2, gnw2, gnb2, gnm2, Wd1, bd1, Wd2, bd2):
    src1d = edge_index[0].astype(jnp.int32)
    dst2d = edge_index[1].astype(jnp.int32).reshape(_NW, _CH, _K)
    batch2d = batch.astype(jnp.int32).reshape(_N, 1)
    zeros128 = jnp.zeros((_NP, _D), jnp.float32)
    ones128 = jnp.ones((_K, _D), jnp.float32)

    degp = _deg_call(dst2d, zeros128, ones128)
    dinv, ht = _prep_call(x, W0, degp)

    layers = [(b0, gnw0, gnb0, gnm0, W1),
              (b1, gnw1, gnb1, gnm1, W2),
              (b2, gnw2, gnb2, gnm2, jnp.zeros((_D, _D), jnp.float32))]
    pools = []
    S0_saved = None
    emb = None
    for (b, gw, gb, gm, Wn) in layers:
        parts = _scatter_call(ht, src1d, dst2d, zeros128)
        t, S = _stats_call(parts, ht, dinv, batch2d, b)
        if S0_saved is None:
            S0_saved = S[0]
        emb, ht, pool = _norm_call(t, S, batch2d, dinv, gw, gb, gm, Wn)
        pools.append(pool)

    z = _head_call(jnp.stack(pools), S0_saved, Wd1, bd1, Wd2, bd2)
    return (emb, z)
